# initial kernel scaffold (unmeasured)
import jax
import jax.numpy as jnp
from jax import lax
from jax.experimental import pallas as pl
from jax.experimental.pallas import tpu as pltpu


def kernel(
    x,
):
    def body(*refs):
        pass

    out_shape = jax.ShapeDtypeStruct(..., jnp.float32)
    return pl.pallas_call(body, out_shape=out_shape)(...)


# baseline (device time: 862973 ns/iter reference)
import jax
import jax.numpy as jnp
from jax import lax
from jax.experimental import pallas as pl
from jax.experimental.pallas import tpu as pltpu

MESH = pl.DeviceIdType.MESH


def kernel(x):
    M, N = x.shape
    HALF = M // 2
    TILE = 1024
    n_tiles = M // TILE

    def body(x_hbm, out_hbm, comm_hbm, ssem1, rsem1, ssem2, rsem2,
             vx, vc, lsems):
        my_x = lax.axis_index("x")
        my_y = lax.axis_index("y")
        y_nbr = (my_x, 1 - my_y)
        x_nbr = (1 - my_x, my_y)

        barrier = pltpu.get_barrier_semaphore()
        for nbr in (y_nbr, x_nbr):
            pl.semaphore_signal(barrier, inc=1, device_id=nbr,
                                device_id_type=MESH)
        pl.semaphore_wait(barrier, 2)

        my_rows = pl.ds(my_x * HALF, HALF)

        rdma1 = pltpu.make_async_remote_copy(
            src_ref=x_hbm.at[my_rows, :],
            dst_ref=comm_hbm.at[my_rows, :],
            send_sem=ssem1,
            recv_sem=rsem1,
            device_id=y_nbr,
            device_id_type=MESH,
        )
        rdma1.start()
        rdma1.wait()

        rdma2 = pltpu.make_async_remote_copy(
            src_ref=comm_hbm.at[my_rows, :],
            dst_ref=comm_hbm.at[my_rows, :],
            send_sem=ssem2,
            recv_sem=rsem2,
            device_id=x_nbr,
            device_id_type=MESH,
        )
        rdma2.start()
        rdma2.wait()

        for t in range(n_tiles):
            rows = pl.ds(t * TILE, TILE)
            ld_x = pltpu.make_async_copy(x_hbm.at[rows, :], vx, lsems.at[0])
            ld_c = pltpu.make_async_copy(comm_hbm.at[rows, :], vc, lsems.at[1])
            ld_x.start()
            ld_c.start()
            ld_x.wait()
            ld_c.wait()
            vx[...] = vx[...] + vc[...]
            st = pltpu.make_async_copy(vx, out_hbm.at[rows, :], lsems.at[2])
            st.start()
            st.wait()

    out, _comm = pl.pallas_call(
        body,
        out_shape=[
            jax.ShapeDtypeStruct((M, N), jnp.float32),
            jax.ShapeDtypeStruct((M, N), jnp.float32),
        ],
        in_specs=[pl.BlockSpec(memory_space=pltpu.HBM)],
        out_specs=[
            pl.BlockSpec(memory_space=pltpu.HBM),
            pl.BlockSpec(memory_space=pltpu.HBM),
        ],
        scratch_shapes=[
            pltpu.SemaphoreType.DMA,
            pltpu.SemaphoreType.DMA,
            pltpu.SemaphoreType.DMA,
            pltpu.SemaphoreType.DMA,
            pltpu.VMEM((TILE, N), jnp.float32),
            pltpu.VMEM((TILE, N), jnp.float32),
            pltpu.SemaphoreType.DMA((3,)),
        ],
        compiler_params=pltpu.CompilerParams(collective_id=0),
    )(x)
    return out


# device time: 437930 ns/iter; 1.9706x vs baseline; 1.9706x over previous
import jax
import jax.numpy as jnp
from jax import lax
from jax.experimental import pallas as pl
from jax.experimental.pallas import tpu as pltpu

MESH = pl.DeviceIdType.MESH

K = 16
LAG = 2


def kernel(x):
    M, N = x.shape
    HALF = M // 2
    C = HALF // K

    def body(x_hbm, out_hbm, comm_hbm, ssem1, rsem1, ssem2, rsem2,
             vxa, vca, vxb, vcb, lsems):
        my_x = lax.axis_index("x")
        my_y = lax.axis_index("y")
        y_nbr = (my_x, 1 - my_y)
        x_nbr = (1 - my_x, my_y)

        barrier = pltpu.get_barrier_semaphore()
        for nbr in (y_nbr, x_nbr):
            pl.semaphore_signal(barrier, inc=1, device_id=nbr,
                                device_id_type=MESH)
        pl.semaphore_wait(barrier, 2)

        def rows_mine(k):
            return pl.ds(my_x * HALF + k * C, C)

        def rows_other(k):
            return pl.ds((1 - my_x) * HALF + k * C, C)

        rdma1 = []
        for k in range(K):
            r = pltpu.make_async_remote_copy(
                src_ref=x_hbm.at[rows_mine(k), :],
                dst_ref=comm_hbm.at[rows_mine(k), :],
                send_sem=ssem1.at[k],
                recv_sem=rsem1.at[k],
                device_id=y_nbr,
                device_id_type=MESH,
            )
            r.start()
            rdma1.append(r)

        rdma2 = []

        def forward(k):
            r = pltpu.make_async_remote_copy(
                src_ref=comm_hbm.at[rows_mine(k), :],
                dst_ref=comm_hbm.at[rows_mine(k), :],
                send_sem=ssem2.at[k],
                recv_sem=rsem2.at[k],
                device_id=x_nbr,
                device_id_type=MESH,
            )
            r.start()
            rdma2.append(r)

        def compute(rows, vx, vc, slot):
            ld_x = pltpu.make_async_copy(x_hbm.at[rows, :], vx,
                                         lsems.at[3 * slot])
            ld_c = pltpu.make_async_copy(comm_hbm.at[rows, :], vc,
                                         lsems.at[3 * slot + 1])
            ld_x.start()
            ld_c.start()
            ld_x.wait()
            ld_c.wait()
            vx[...] = vx[...] + vc[...]
            st = pltpu.make_async_copy(vx, out_hbm.at[rows, :],
                                       lsems.at[3 * slot + 2])
            st.start()
            st.wait()

        for k in range(K):
            rdma1[k].wait_recv()
            forward(k)
            compute(rows_mine(k), vxa, vca, 0)
            if k >= LAG:
                rdma2[k - LAG].wait_recv()
                compute(rows_other(k - LAG), vxb, vcb, 1)

        for k in range(K - LAG, K):
            rdma2[k].wait_recv()
            compute(rows_other(k), vxb, vcb, 1)

        for k in range(K):
            rdma1[k].wait_send()
            rdma2[k].wait_send()

    out, _comm = pl.pallas_call(
        body,
        out_shape=[
            jax.ShapeDtypeStruct((M, N), jnp.float32),
            jax.ShapeDtypeStruct((M, N), jnp.float32),
        ],
        in_specs=[pl.BlockSpec(memory_space=pltpu.HBM)],
        out_specs=[
            pl.BlockSpec(memory_space=pltpu.HBM),
            pl.BlockSpec(memory_space=pltpu.HBM),
        ],
        scratch_shapes=[
            pltpu.SemaphoreType.DMA((K,)),
            pltpu.SemaphoreType.DMA((K,)),
            pltpu.SemaphoreType.DMA((K,)),
            pltpu.SemaphoreType.DMA((K,)),
            pltpu.VMEM((C, N), jnp.float32),
            pltpu.VMEM((C, N), jnp.float32),
            pltpu.VMEM((C, N), jnp.float32),
            pltpu.VMEM((C, N), jnp.float32),
            pltpu.SemaphoreType.DMA((6,)),
        ],
        compiler_params=pltpu.CompilerParams(collective_id=0),
    )(x)
    return out


# device time: 437657 ns/iter; 1.9718x vs baseline; 1.0006x over previous
import jax
import jax.numpy as jnp
from jax import lax
from jax.experimental import pallas as pl
from jax.experimental.pallas import tpu as pltpu

MESH = pl.DeviceIdType.MESH

K = 16
LAG = 2


def kernel(x):
    M, N = x.shape
    HALF = M // 2
    C = HALF // K

    def body(x_hbm, out_hbm, comm_hbm, ssem1, rsem1, ssem2, rsem2,
             vx, vc, vo, lx, lc, st):
        my_x = lax.axis_index("x")
        my_y = lax.axis_index("y")
        y_nbr = (my_x, 1 - my_y)
        x_nbr = (1 - my_x, my_y)

        barrier = pltpu.get_barrier_semaphore()
        for nbr in (y_nbr, x_nbr):
            pl.semaphore_signal(barrier, inc=1, device_id=nbr,
                                device_id_type=MESH)
        pl.semaphore_wait(barrier, 2)

        def rows_mine(k):
            return pl.ds(my_x * HALF + k * C, C)

        def rows_other(k):
            return pl.ds((1 - my_x) * HALF + k * C, C)

        rdma1 = []
        for k in range(K):
            r = pltpu.make_async_remote_copy(
                src_ref=x_hbm.at[rows_mine(k), :],
                dst_ref=comm_hbm.at[rows_mine(k), :],
                send_sem=ssem1.at[k],
                recv_sem=rsem1.at[k],
                device_id=y_nbr,
                device_id_type=MESH,
            )
            r.start()
            rdma1.append(r)

        rdma2 = []

        def forward(k):
            r = pltpu.make_async_remote_copy(
                src_ref=comm_hbm.at[rows_mine(k), :],
                dst_ref=comm_hbm.at[rows_mine(k), :],
                send_sem=ssem2.at[k],
                recv_sem=rsem2.at[k],
                device_id=x_nbr,
                device_id_type=MESH,
            )
            r.start()
            rdma2.append(r)

        tasks = []
        for k in range(K):
            tasks.append(("p1", k))
            if k >= LAG:
                tasks.append(("p2", k - LAG))
        for k in range(K - LAG, K):
            tasks.append(("p2", k))

        def task_rows(t):
            phase, k = tasks[t]
            return rows_mine(k) if phase == "p1" else rows_other(k)

        def start_xload(t):
            s = t % 2
            pltpu.make_async_copy(x_hbm.at[task_rows(t), :], vx.at[s],
                                  lx.at[s]).start()

        start_xload(0)
        for t in range(len(tasks)):
            s = t % 2
            phase, k = tasks[t]
            if phase == "p1":
                rdma1[k].wait_recv()
                forward(k)
            else:
                rdma2[k].wait_recv()
            ld_c = pltpu.make_async_copy(comm_hbm.at[task_rows(t), :],
                                         vc.at[s], lc.at[s])
            ld_c.start()
            if t + 1 < len(tasks):
                start_xload(t + 1)
            pltpu.make_async_copy(x_hbm.at[task_rows(t), :], vx.at[s],
                                  lx.at[s]).wait()
            ld_c.wait()
            if t >= 2:
                pltpu.make_async_copy(vo.at[s], out_hbm.at[task_rows(t - 2), :],
                                      st.at[s]).wait()
            vo[s] = vx[s] + vc[s]
            pltpu.make_async_copy(vo.at[s], out_hbm.at[task_rows(t), :],
                                  st.at[s]).start()

        n_t = len(tasks)
        for t in (n_t - 2, n_t - 1):
            s = t % 2
            pltpu.make_async_copy(vo.at[s], out_hbm.at[task_rows(t), :],
                                  st.at[s]).wait()
        for k in range(K):
            rdma1[k].wait_send()
            rdma2[k].wait_send()

    out, _comm = pl.pallas_call(
        body,
        out_shape=[
            jax.ShapeDtypeStruct((M, N), jnp.float32),
            jax.ShapeDtypeStruct((M, N), jnp.float32),
        ],
        in_specs=[pl.BlockSpec(memory_space=pltpu.HBM)],
        out_specs=[
            pl.BlockSpec(memory_space=pltpu.HBM),
            pl.BlockSpec(memory_space=pltpu.HBM),
        ],
        scratch_shapes=[
            pltpu.SemaphoreType.DMA((K,)),
            pltpu.SemaphoreType.DMA((K,)),
            pltpu.SemaphoreType.DMA((K,)),
            pltpu.SemaphoreType.DMA((K,)),
            pltpu.VMEM((2, C, N), jnp.float32),
            pltpu.VMEM((2, C, N), jnp.float32),
            pltpu.VMEM((2, C, N), jnp.float32),
            pltpu.SemaphoreType.DMA((2,)),
            pltpu.SemaphoreType.DMA((2,)),
            pltpu.SemaphoreType.DMA((2,)),
        ],
        compiler_params=pltpu.CompilerParams(collective_id=0),
    )(x)
    return out


# device time: 412801 ns/iter; 2.0905x vs baseline; 1.0602x over previous
import jax
import jax.numpy as jnp
from jax import lax
from jax.experimental import pallas as pl
from jax.experimental.pallas import tpu as pltpu

MESH = pl.DeviceIdType.MESH

K = 16
LAG = 2


def kernel(x):
    M, N = x.shape
    HALF = M // 2
    C = HALF // K

    def body(x_hbm, out_hbm, comm_hbm, ssem1, rsem1, ssem2, rsem2,
             vx, vc, vo, lx, lc, st):
        my_x = lax.axis_index("x")
        my_y = lax.axis_index("y")
        y_nbr = (my_x, 1 - my_y)
        x_nbr = (1 - my_x, my_y)

        barrier = pltpu.get_barrier_semaphore()
        for nbr in (y_nbr, x_nbr):
            pl.semaphore_signal(barrier, inc=1, device_id=nbr,
                                device_id_type=MESH)
        pl.semaphore_wait(barrier, 2)

        def rows_mine(k):
            return pl.ds(my_x * HALF + k * C, C)

        def rows_other(k):
            return pl.ds((1 - my_x) * HALF + k * C, C)

        rdma1 = []
        for k in range(K):
            r = pltpu.make_async_remote_copy(
                src_ref=x_hbm.at[rows_mine(k), :],
                dst_ref=comm_hbm.at[rows_mine(k), :],
                send_sem=ssem1.at[k],
                recv_sem=rsem1.at[k],
                device_id=y_nbr,
                device_id_type=MESH,
            )
            r.start()
            rdma1.append(r)

        rdma2 = []

        def forward(k):
            r = pltpu.make_async_remote_copy(
                src_ref=comm_hbm.at[rows_mine(k), :],
                dst_ref=comm_hbm.at[rows_mine(k), :],
                send_sem=ssem2.at[k],
                recv_sem=rsem2.at[k],
                device_id=x_nbr,
                device_id_type=MESH,
            )
            r.start()
            rdma2.append(r)

        import os
        _DIAG = os.environ.get("DIAG_PHASE1_ONLY") == "1"
        tasks = []
        for k in range(K):
            tasks.append(("p1", k))
            if not _DIAG and k >= LAG:
                tasks.append(("p2", k - LAG))
        if not _DIAG:
            for k in range(K - LAG, K):
                tasks.append(("p2", k))

        def task_rows(t):
            phase, k = tasks[t]
            return rows_mine(k) if phase == "p1" else rows_other(k)

        def start_xload(t):
            s = t % 2
            pltpu.make_async_copy(x_hbm.at[task_rows(t), :], vx.at[s],
                                  lx.at[s]).start()

        start_xload(0)
        for t in range(len(tasks)):
            s = t % 2
            phase, k = tasks[t]
            if phase == "p1":
                rdma1[k].wait_recv()
                if not _DIAG:
                    forward(k)
            else:
                rdma2[k].wait_recv()
            ld_c = pltpu.make_async_copy(comm_hbm.at[task_rows(t), :],
                                         vc.at[s], lc.at[s])
            ld_c.start()
            if t + 1 < len(tasks):
                start_xload(t + 1)
            pltpu.make_async_copy(x_hbm.at[task_rows(t), :], vx.at[s],
                                  lx.at[s]).wait()
            ld_c.wait()
            if t >= 2:
                pltpu.make_async_copy(vo.at[s], out_hbm.at[task_rows(t - 2), :],
                                      st.at[s]).wait()
            vo[s] = vx[s] + vc[s]
            pltpu.make_async_copy(vo.at[s], out_hbm.at[task_rows(t), :],
                                  st.at[s]).start()

        n_t = len(tasks)
        for t in (n_t - 2, n_t - 1):
            s = t % 2
            pltpu.make_async_copy(vo.at[s], out_hbm.at[task_rows(t), :],
                                  st.at[s]).wait()
        for k in range(K):
            rdma1[k].wait_send()
            if not _DIAG:
                rdma2[k].wait_send()

    out, _comm = pl.pallas_call(
        body,
        out_shape=[
            jax.ShapeDtypeStruct((M, N), jnp.float32),
            jax.ShapeDtypeStruct((M, N), jnp.float32),
        ],
        in_specs=[pl.BlockSpec(memory_space=pltpu.HBM)],
        out_specs=[
            pl.BlockSpec(memory_space=pltpu.HBM),
            pl.BlockSpec(memory_space=pltpu.HBM),
        ],
        scratch_shapes=[
            pltpu.SemaphoreType.DMA((K,)),
            pltpu.SemaphoreType.DMA((K,)),
            pltpu.SemaphoreType.DMA((K,)),
            pltpu.SemaphoreType.DMA((K,)),
            pltpu.VMEM((2, C, N), jnp.float32),
            pltpu.VMEM((2, C, N), jnp.float32),
            pltpu.VMEM((2, C, N), jnp.float32),
            pltpu.SemaphoreType.DMA((2,)),
            pltpu.SemaphoreType.DMA((2,)),
            pltpu.SemaphoreType.DMA((2,)),
        ],
        compiler_params=pltpu.CompilerParams(collective_id=0),
    )(x)
    return out
